# baseline (speedup 1.0000x reference)
import functools

import jax
import jax.numpy as jnp
from jax import lax
from jax.experimental import pallas as pl
from jax.experimental.pallas import tpu as pltpu
from jax.experimental.pallas import tpu_sc as plsc

_N = 8192
_D = 256
_K = 8192
_NB = _N // 2
_T = 512
_NT = _NB // _T

_H = 128
_NH = _T // _H
_S = 128
_NS = _K // _S


def _dist_argmin_body(z2_ref, wt_ref, idx_ref, acc_out_ref, wsq_ref, acc_ref):
    i = pl.program_id(0)
    z2 = z2_ref[...]

    @pl.when(i == 0)
    def _():
        wt = wt_ref[...]
        wsq_ref[...] = jnp.sum(wt * wt, axis=0, keepdims=True)
        acc_ref[0] = 0.0

    zsq = 0.25 * jnp.sum(z2 * z2, axis=1, keepdims=True)
    s2 = lax.dot_general(z2, wt_ref[...], (((1,), (0,)), ((), ())),
                         preferred_element_type=jnp.float32)
    wsq = wsq_ref[...]

    idx_parts = []
    min_parts = []
    for h in range(_NH):
        zsq_h = lax.slice(zsq, (h * _H, 0), ((h + 1) * _H, 1))
        run_min = None
        run_col = None
        for s in range(_NS):
            wsq_s = lax.slice(wsq, (0, s * _S), (1, (s + 1) * _S))
            s2_s = lax.slice(s2, (h * _H, s * _S),
                             ((h + 1) * _H, (s + 1) * _S))
            d_s = (zsq_h + wsq_s) + s2_s
            if s == 0:
                run_min = d_s
                run_col = jnp.zeros((_H, _S), jnp.int32)
            else:
                better = d_s < run_min
                run_min = jnp.where(better, d_s, run_min)
                run_col = jnp.where(better, jnp.int32(s), run_col)
        lmin = jnp.min(run_min, axis=1, keepdims=True)
        gidx = run_col * _S + lax.broadcasted_iota(jnp.int32, (_H, _S), 1)
        cand = jnp.where(run_min == lmin, gidx, 2**31 - 1)
        idx_parts.append(jnp.min(cand, axis=1, keepdims=True))
        min_parts.append(lmin)
    idx_ref[...] = jnp.concatenate(idx_parts, axis=0)

    acc_ref[0] += jnp.sum(jnp.concatenate(min_parts, axis=0))

    @pl.when(i == pl.num_programs(0) - 1)
    def _():
        acc_out_ref[...] = jnp.full((1, 1), acc_ref[0], jnp.float32)


def _dist_argmin(z_flat, wt):
    return pl.pallas_call(
        _dist_argmin_body,
        grid=(_NT,),
        in_specs=[
            pl.BlockSpec((_T, _D), lambda i: (i, 0)),
            pl.BlockSpec((_D, _K), lambda i: (0, 0)),
        ],
        out_specs=[
            pl.BlockSpec((_T, 1), lambda i: (i, 0)),
            pl.BlockSpec((1, 1), lambda i: (0, 0)),
        ],
        out_shape=[
            jax.ShapeDtypeStruct((_NB, 1), jnp.int32),
            jax.ShapeDtypeStruct((1, 1), jnp.float32),
        ],
        scratch_shapes=[pltpu.VMEM((1, _K), jnp.float32),
                        pltpu.SMEM((1,), jnp.float32)],
    )(z_flat, wt)


_NW = 32
_ROWS_PER_W = _NB // _NW
_CHUNK = 128
_NCHUNK = _ROWS_PER_W // _CHUNK


@functools.lru_cache(maxsize=1)
def _make_sc_gather():
    mesh = plsc.VectorSubcoreMesh(
        core_axis_name="c", subcore_axis_name="s", num_cores=2)

    @functools.partial(
        pl.kernel,
        mesh=mesh,
        out_type=jax.ShapeDtypeStruct((_NB, _D), jnp.float32),
        scratch_types=[
            pltpu.VMEM((_NCHUNK, _CHUNK), jnp.int32),
            pltpu.VMEM((_ROWS_PER_W, _D), jnp.float32),
            pltpu.SemaphoreType.DMA,
        ],
    )
    def gather(w_hbm, idx_hbm, out_hbm, idx_v, rows_v, sem):
        wid = lax.axis_index("s") * 2 + lax.axis_index("c")
        pltpu.sync_copy(idx_hbm.at[pl.ds(wid * _NCHUNK, _NCHUNK)], idx_v)
        copies = [
            pltpu.async_copy(
                w_hbm.at[idx_v.at[b]],
                rows_v.at[pl.ds(b * _CHUNK, _CHUNK)],
                sem,
            )
            for b in range(_NCHUNK)
        ]
        for c in copies:
            c.wait()
        pltpu.sync_copy(rows_v, out_hbm.at[pl.ds(wid * _ROWS_PER_W, _ROWS_PER_W)])

    return gather


def kernel(z, W):
    b, d, h, w = z.shape
    hb = b // 2
    z_flat = (-2.0) * jnp.transpose(z, (0, 2, 3, 1)).reshape(_N, _D)
    wt = W.T
    sc_gather = _make_sc_gather()

    halves = []
    accs = []
    idxs = []
    for half in range(2):
        zf = lax.slice(z_flat, (half * _NB, 0), ((half + 1) * _NB, _D))
        idx2d, acc = _dist_argmin(zf, wt)
        idx = idx2d.reshape(_NB)
        zq_flat = sc_gather(W, idx.reshape(_NB // _CHUNK, _CHUNK))
        z_q_half = zq_flat.reshape(hb, h, w, d).transpose(0, 3, 1, 2)
        halves.append(z_q_half)
        accs.append(acc.reshape(()))
        idxs.append(idx)

    loss = (accs[0] + accs[1]) * (1.25 / (_N * _D))
    z_q = jnp.concatenate(halves, axis=0)
    idx = jnp.concatenate(idxs, axis=0)
    return z_q, idx, loss


# speedup vs baseline: 1.1819x; 1.1819x over previous
import functools

import jax
import jax.numpy as jnp
from jax import lax
from jax.experimental import pallas as pl
from jax.experimental.pallas import tpu as pltpu
from jax.experimental.pallas import tpu_sc as plsc

_N = 8192
_D = 256
_K = 8192
_NB = _N
_T = 512
_NT = _NB // _T

_H = 64
_NH = _T // _H
_S = 128
_NS = _K // _S


def _dist_argmin_body(z_ref, wt_ref, idx_ref, acc_out_ref, wsq_ref, acc_ref):
    i = pl.program_id(0)
    z2 = (-2.0) * jnp.transpose(z_ref[0], (1, 0))

    @pl.when(i == 0)
    def _():
        wt = wt_ref[...]
        wsq_ref[...] = jnp.sum(wt * wt, axis=0, keepdims=True)
        acc_ref[0] = 0.0

    zsq = 0.25 * jnp.sum(z2 * z2, axis=1, keepdims=True)
    s2 = lax.dot_general(z2, wt_ref[...], (((1,), (0,)), ((), ())),
                         preferred_element_type=jnp.float32)
    wsq = wsq_ref[...]

    idx_parts = []
    min_parts = []
    for h in range(_NH):
        zsq_h = lax.slice(zsq, (h * _H, 0), ((h + 1) * _H, 1))
        run_min = None
        run_col = None
        for s in range(_NS):
            wsq_s = lax.slice(wsq, (0, s * _S), (1, (s + 1) * _S))
            s2_s = lax.slice(s2, (h * _H, s * _S),
                             ((h + 1) * _H, (s + 1) * _S))
            d_s = (zsq_h + wsq_s) + s2_s
            if s == 0:
                run_min = d_s
                run_col = jnp.zeros((_H, _S), jnp.int32)
            else:
                better = d_s < run_min
                run_min = jnp.where(better, d_s, run_min)
                run_col = jnp.where(better, jnp.int32(s), run_col)
        lmin = jnp.min(run_min, axis=1, keepdims=True)
        gidx = run_col * _S + lax.broadcasted_iota(jnp.int32, (_H, _S), 1)
        cand = jnp.where(run_min == lmin, gidx, 2**31 - 1)
        idx_parts.append(jnp.min(cand, axis=1, keepdims=True))
        min_parts.append(lmin)
    idx_ref[...] = jnp.concatenate(idx_parts, axis=0)

    acc_ref[0] += jnp.sum(jnp.concatenate(min_parts, axis=0))

    @pl.when(i == pl.num_programs(0) - 1)
    def _():
        acc_out_ref[...] = jnp.full((1, 1), acc_ref[0], jnp.float32)


def _dist_argmin(z_flat, wt):
    return pl.pallas_call(
        _dist_argmin_body,
        grid=(_NT,),
        in_specs=[
            pl.BlockSpec((1, _D, _T), lambda i: (i // 2, 0, i % 2)),
            pl.BlockSpec((_D, _K), lambda i: (0, 0)),
        ],
        out_specs=[
            pl.BlockSpec((_T, 1), lambda i: (i, 0)),
            pl.BlockSpec((1, 1), lambda i: (0, 0)),
        ],
        out_shape=[
            jax.ShapeDtypeStruct((_NB, 1), jnp.int32),
            jax.ShapeDtypeStruct((1, 1), jnp.float32),
        ],
        scratch_shapes=[pltpu.VMEM((1, _K), jnp.float32),
                        pltpu.SMEM((1,), jnp.float32)],
    )(z_flat, wt)


_NW = 32
_ROWS_PER_W = _NB // _NW
_CHUNK = 128
_NCHUNK = _ROWS_PER_W // _CHUNK


@functools.lru_cache(maxsize=1)
def _make_sc_gather():
    mesh = plsc.VectorSubcoreMesh(
        core_axis_name="c", subcore_axis_name="s", num_cores=2)

    @functools.partial(
        pl.kernel,
        mesh=mesh,
        out_type=jax.ShapeDtypeStruct((_NB, _D), jnp.float32),
        scratch_types=[
            pltpu.VMEM((_NCHUNK, _CHUNK), jnp.int32),
            pltpu.VMEM((_ROWS_PER_W, _D), jnp.float32),
            pltpu.SemaphoreType.DMA,
        ],
    )
    def gather(w_hbm, idx_hbm, out_hbm, idx_v, rows_v, sem):
        wid = lax.axis_index("s") * 2 + lax.axis_index("c")
        pltpu.sync_copy(idx_hbm.at[pl.ds(wid * _NCHUNK, _NCHUNK)], idx_v)
        copies = [
            pltpu.async_copy(
                w_hbm.at[idx_v.at[b]],
                rows_v.at[pl.ds(b * _CHUNK, _CHUNK)],
                sem,
            )
            for b in range(_NCHUNK)
        ]
        for c in copies:
            c.wait()
        pltpu.sync_copy(rows_v, out_hbm.at[pl.ds(wid * _ROWS_PER_W, _ROWS_PER_W)])

    return gather


def kernel(z, W):
    b, d, h, w = z.shape
    z3 = z.reshape(b, d, h * w)
    wt = W.T
    idx2d, acc = _dist_argmin(z3, wt)
    idx = idx2d.reshape(_N)
    zq_flat = _make_sc_gather()(W, idx.reshape(_N // _CHUNK, _CHUNK))
    z_q = zq_flat.reshape(b, h, w, d).transpose(0, 3, 1, 2)
    loss = acc.reshape(()) * (1.25 / (_N * _D))
    return z_q, idx, loss


# speedup vs baseline: 1.2470x; 1.0551x over previous
import functools

import jax
import jax.numpy as jnp
from jax import lax
from jax.experimental import pallas as pl
from jax.experimental.pallas import tpu as pltpu
from jax.experimental.pallas import tpu_sc as plsc

_N = 8192
_D = 256
_K = 8192
_NB = _N
_T = 1024
_NT = _NB // _T

_H = 64
_NH = _T // _H
_S = 128
_NS = _K // _S


def _dist_argmin_body(z2_ref, wt_ref, idx_ref, acc_out_ref, wsq_ref, acc_ref):
    i = pl.program_id(0)
    z2 = z2_ref[...]

    @pl.when(i == 0)
    def _():
        wt = wt_ref[...]
        wsq_ref[...] = jnp.sum(wt * wt, axis=0, keepdims=True)
        acc_ref[0] = 0.0

    zsq = 0.25 * jnp.sum(z2 * z2, axis=1, keepdims=True)
    s2 = lax.dot_general(z2, wt_ref[...], (((1,), (0,)), ((), ())),
                         preferred_element_type=jnp.float32)
    wsq = wsq_ref[...]

    idx_parts = []
    min_parts = []
    for h in range(_NH):
        zsq_h = lax.slice(zsq, (h * _H, 0), ((h + 1) * _H, 1))
        run_min = None
        run_col = None
        for s in range(_NS):
            wsq_s = lax.slice(wsq, (0, s * _S), (1, (s + 1) * _S))
            s2_s = lax.slice(s2, (h * _H, s * _S),
                             ((h + 1) * _H, (s + 1) * _S))
            d_s = (zsq_h + wsq_s) + s2_s
            if s == 0:
                run_min = d_s
                run_col = jnp.zeros((_H, _S), jnp.int32)
            else:
                better = d_s < run_min
                run_min = jnp.where(better, d_s, run_min)
                run_col = jnp.where(better, jnp.int32(s), run_col)
        lmin = jnp.min(run_min, axis=1, keepdims=True)
        gidx = run_col * _S + lax.broadcasted_iota(jnp.int32, (_H, _S), 1)
        cand = jnp.where(run_min == lmin, gidx, 2**31 - 1)
        idx_parts.append(jnp.min(cand, axis=1, keepdims=True))
        min_parts.append(lmin)
    idx_ref[...] = jnp.concatenate(idx_parts, axis=0)

    acc_ref[0] += jnp.sum(jnp.concatenate(min_parts, axis=0))

    @pl.when(i == pl.num_programs(0) - 1)
    def _():
        acc_out_ref[...] = jnp.full((1, 1), acc_ref[0], jnp.float32)


def _dist_argmin(z_flat, wt):
    return pl.pallas_call(
        _dist_argmin_body,
        grid=(_NT,),
        in_specs=[
            pl.BlockSpec((_T, _D), lambda i: (i, 0)),
            pl.BlockSpec((_D, _K), lambda i: (0, 0)),
        ],
        out_specs=[
            pl.BlockSpec((_T, 1), lambda i: (i, 0)),
            pl.BlockSpec((1, 1), lambda i: (0, 0)),
        ],
        out_shape=[
            jax.ShapeDtypeStruct((_NB, 1), jnp.int32),
            jax.ShapeDtypeStruct((1, 1), jnp.float32),
        ],
        scratch_shapes=[pltpu.VMEM((1, _K), jnp.float32),
                        pltpu.SMEM((1,), jnp.float32)],
    )(z_flat, wt)


_NW = 32
_ROWS_PER_W = _NB // _NW
_CHUNK = 128
_NCHUNK = _ROWS_PER_W // _CHUNK


@functools.lru_cache(maxsize=1)
def _make_sc_gather():
    mesh = plsc.VectorSubcoreMesh(
        core_axis_name="c", subcore_axis_name="s", num_cores=2)

    @functools.partial(
        pl.kernel,
        mesh=mesh,
        out_type=jax.ShapeDtypeStruct((_NB, _D), jnp.float32),
        scratch_types=[
            pltpu.VMEM((_NCHUNK, _CHUNK), jnp.int32),
            pltpu.VMEM((_ROWS_PER_W, _D), jnp.float32),
            pltpu.SemaphoreType.DMA,
        ],
    )
    def gather(w_hbm, idx_hbm, out_hbm, idx_v, rows_v, sem):
        wid = lax.axis_index("s") * 2 + lax.axis_index("c")
        pltpu.sync_copy(idx_hbm.at[pl.ds(wid * _NCHUNK, _NCHUNK)], idx_v)
        copies = [
            pltpu.async_copy(
                w_hbm.at[idx_v.at[b]],
                rows_v.at[pl.ds(b * _CHUNK, _CHUNK)],
                sem,
            )
            for b in range(_NCHUNK)
        ]
        for c in copies:
            c.wait()
        pltpu.sync_copy(rows_v, out_hbm.at[pl.ds(wid * _ROWS_PER_W, _ROWS_PER_W)])

    return gather


def kernel(z, W):
    b, d, h, w = z.shape
    z_flat = (-2.0) * jnp.transpose(z, (0, 2, 3, 1)).reshape(_N, _D)
    wt = W.T
    idx2d, acc = _dist_argmin(z_flat, wt)
    idx = idx2d.reshape(_N)
    zq_flat = _make_sc_gather()(W, idx.reshape(_N // _CHUNK, _CHUNK))
    z_q = zq_flat.reshape(b, h, w, d).transpose(0, 3, 1, 2)
    loss = acc.reshape(()) * (1.25 / (_N * _D))
    return z_q, idx, loss
